# Initial kernel scaffold; baseline (speedup 1.0000x reference)
import jax
import jax.numpy as jnp
from jax.experimental import pallas as pl


def kernel(encoder_input_embed, encoder_input_mask, ocr_emb, common_voc_emb, prev_inds, pos_emb, type_emb, cv_gamma, cv_beta, ocr_gamma, ocr_beta, emb_gamma, emb_beta):
    raise NotImplementedError("write your pallas kernel here")


# speedup vs baseline: 3.6613x; 3.6613x over previous
import functools

import jax
import jax.numpy as jnp
from jax import lax
from jax.experimental import pallas as pl
from jax.experimental.pallas import tpu as pltpu
from jax.experimental.pallas import tpu_sc as plsc

_V = 100000
_OCR = 50
_L = 200
_H = 64
_NW = 32
_RPW = 64
_TOT = _NW * _RPW
_EPS = 1e-5
_LANE = 16


def _rsqrt16(a_scalar):
    av = jnp.full((_LANE,), a_scalar, dtype=jnp.float32)
    yi = lax.bitcast_convert_type(av, jnp.int32)
    yi = jnp.int32(0x5F3759DF) - lax.shift_right_logical(yi, 1)
    y = lax.bitcast_convert_type(yi, jnp.float32)
    for _ in range(4):
        y = y * (1.5 - 0.5 * av * y * y)
    return y


def _sc_body(idx_hbm, obase_hbm, ipos_hbm, cv_hbm, ocr_hbm, pos_hbm, type_hbm,
             cvg_hbm, cvb_hbm, ocg_hbm, ocb_hbm, eg_hbm, eb_hbm,
             out_hbm,
             idx_v, ob_v, icv_v, ioc_v, ipos_v, sel_v,
             cv_rows, oc_rows, pos_rows, out_rows,
             cvg_v, cvb_v, ocg_v, ocb_v, eg_v, eb_v, ty0_v,
             sem0, sem1, sem2):
    wid = lax.axis_index("s") * 2 + lax.axis_index("c")
    base = wid * _RPW

    pltpu.sync_copy(idx_hbm.at[pl.ds(base, _RPW)], idx_v)
    pltpu.sync_copy(obase_hbm.at[pl.ds(base, _RPW)], ob_v)
    pltpu.sync_copy(ipos_hbm.at[pl.ds(base, _RPW)], ipos_v)
    pltpu.sync_copy(cvg_hbm, cvg_v)
    pltpu.sync_copy(cvb_hbm, cvb_v)
    pltpu.sync_copy(ocg_hbm, ocg_v)
    pltpu.sync_copy(ocb_hbm, ocb_v)
    pltpu.sync_copy(eg_hbm, eg_v)
    pltpu.sync_copy(eb_hbm, eb_v)
    pltpu.sync_copy(type_hbm.at[0], ty0_v)

    for j in range(_RPW // _LANE):
        sl = pl.ds(j * _LANE, _LANE)
        v = idx_v[sl]
        is_cv = v < _V
        icv_v[sl] = jnp.where(is_cv, v, 0)
        ioc_v[sl] = jnp.where(is_cv, 0, ob_v[sl] + (v - _V))
        sel_v[sl] = jnp.where(is_cv, jnp.float32(1.0), jnp.float32(0.0))

    c1 = pltpu.async_copy(cv_hbm.at[icv_v], cv_rows, sem0)
    c2 = pltpu.async_copy(ocr_hbm.at[ioc_v], oc_rows, sem1)
    c3 = pltpu.async_copy(pos_hbm.at[ipos_v], pos_rows, sem2)
    c1.wait()
    c2.wait()
    c3.wait()

    def row_body(r, carry):
        rb = jnp.full((_LANE,), r, dtype=jnp.int32)
        sel = plsc.load_gather(sel_v, [rb])
        inv = 1.0 - sel
        xs = []
        ps = []
        for h in range(_H // _LANE):
            hs = pl.ds(h * _LANE, _LANE)
            xs.append(cv_rows[r, hs] * sel + oc_rows[r, hs] * inv)
            ps.append(pos_rows[r, hs] + ty0_v[hs])
        mu = (jnp.sum(xs[0]) + jnp.sum(xs[1]) + jnp.sum(xs[2]) + jnp.sum(xs[3])) * (1.0 / _H)
        mup = (jnp.sum(ps[0]) + jnp.sum(ps[1]) + jnp.sum(ps[2]) + jnp.sum(ps[3])) * (1.0 / _H)
        dx = [x - mu for x in xs]
        dp = [p - mup for p in ps]
        var = (jnp.sum(dx[0] * dx[0]) + jnp.sum(dx[1] * dx[1])
               + jnp.sum(dx[2] * dx[2]) + jnp.sum(dx[3] * dx[3])) * (1.0 / _H)
        varp = (jnp.sum(dp[0] * dp[0]) + jnp.sum(dp[1] * dp[1])
                + jnp.sum(dp[2] * dp[2]) + jnp.sum(dp[3] * dp[3])) * (1.0 / _H)
        rstd = _rsqrt16(var + _EPS)
        rstdp = _rsqrt16(varp + _EPS)
        for h in range(_H // _LANE):
            hs = pl.ds(h * _LANE, _LANE)
            g = cvg_v[hs] * sel + ocg_v[hs] * inv
            be = cvb_v[hs] * sel + ocb_v[hs] * inv
            out_rows[r, hs] = (dx[h] * rstd * g + be
                               + dp[h] * rstdp * eg_v[hs] + eb_v[hs])
        return carry

    lax.fori_loop(0, _RPW, row_body, 0)
    pltpu.sync_copy(out_rows, out_hbm.at[pl.ds(base, _RPW)])


@functools.partial(
    pl.kernel,
    mesh=plsc.VectorSubcoreMesh(core_axis_name="c", subcore_axis_name="s"),
    out_type=jax.ShapeDtypeStruct((_TOT, _H), jnp.float32),
    compiler_params=pltpu.CompilerParams(
        use_tc_tiling_on_sc=False, needs_layout_passes=False),
    scratch_types=[
        pltpu.VMEM((_RPW,), jnp.int32),
        pltpu.VMEM((_RPW,), jnp.int32),
        pltpu.VMEM((_RPW,), jnp.int32),
        pltpu.VMEM((_RPW,), jnp.int32),
        pltpu.VMEM((_RPW,), jnp.int32),
        pltpu.VMEM((_RPW,), jnp.float32),
        pltpu.VMEM((_RPW, _H), jnp.float32),
        pltpu.VMEM((_RPW, _H), jnp.float32),
        pltpu.VMEM((_RPW, _H), jnp.float32),
        pltpu.VMEM((_RPW, _H), jnp.float32),
        pltpu.VMEM((_H,), jnp.float32),
        pltpu.VMEM((_H,), jnp.float32),
        pltpu.VMEM((_H,), jnp.float32),
        pltpu.VMEM((_H,), jnp.float32),
        pltpu.VMEM((_H,), jnp.float32),
        pltpu.VMEM((_H,), jnp.float32),
        pltpu.VMEM((_H,), jnp.float32),
        pltpu.SemaphoreType.DMA,
        pltpu.SemaphoreType.DMA,
        pltpu.SemaphoreType.DMA,
    ],
)
def _prev_embed_sc(idx_hbm, obase_hbm, ipos_hbm, cv_hbm, ocr_hbm, pos_hbm,
                   type_hbm, cvg_hbm, cvb_hbm, ocg_hbm, ocb_hbm, eg_hbm,
                   eb_hbm, out_hbm, *scratch):
    _sc_body(idx_hbm, obase_hbm, ipos_hbm, cv_hbm, ocr_hbm, pos_hbm, type_hbm,
             cvg_hbm, cvb_hbm, ocg_hbm, ocb_hbm, eg_hbm, eb_hbm,
             out_hbm, *scratch)


def kernel(encoder_input_embed, encoder_input_mask, ocr_emb, common_voc_emb,
           prev_inds, pos_emb, type_emb, cv_gamma, cv_beta, ocr_gamma,
           ocr_beta, emb_gamma, emb_beta):
    b, l = prev_inds.shape
    idx_flat = prev_inds.reshape(-1).astype(jnp.int32)
    idx_pad = jnp.zeros((_TOT,), jnp.int32).at[: b * l].set(idx_flat)
    flat = jnp.arange(_TOT, dtype=jnp.int32)
    obase = (flat // _L) * _OCR
    ipos = jnp.remainder(flat, _L)
    ocr_flat = ocr_emb.reshape(-1, _H)

    prev = _prev_embed_sc(idx_pad, obase, ipos, common_voc_emb, ocr_flat,
                          pos_emb, type_emb, cv_gamma, cv_beta, ocr_gamma,
                          ocr_beta, emb_gamma, emb_beta)
    prev_embed = prev[: b * l].reshape(b, l, _H)

    encoder_inputs = jnp.concatenate([encoder_input_embed, prev_embed], axis=1)
    encoder_inputs_mask = jnp.concatenate(
        [encoder_input_mask, jnp.zeros((b, l), jnp.float32)], axis=1)
    ext = jnp.full((b, 1, l, l), -10000.0, jnp.float32)
    return (encoder_inputs, encoder_inputs_mask, ext)


# speedup vs baseline: 3.6891x; 1.0076x over previous
import functools

import jax
import jax.numpy as jnp
from jax import lax
from jax.experimental import pallas as pl
from jax.experimental.pallas import tpu as pltpu
from jax.experimental.pallas import tpu_sc as plsc

_V = 100000
_OCR = 50
_L = 200
_H = 64
_NW = 32
_RPW = 64
_TOT = _NW * _RPW
_EPS = 1e-5
_LANE = 16


def _rsqrt16(a_scalar):
    av = jnp.full((_LANE,), a_scalar, dtype=jnp.float32)
    yi = lax.bitcast_convert_type(av, jnp.int32)
    yi = jnp.int32(0x5F3759DF) - lax.shift_right_logical(yi, 1)
    y = lax.bitcast_convert_type(yi, jnp.float32)
    for _ in range(4):
        y = y * (1.5 - 0.5 * av * y * y)
    return y


def _sc_body(idx_hbm, obase_hbm, ipos_hbm, cv_hbm, ocr_hbm, pos_hbm, type_hbm,
             cvg_hbm, cvb_hbm, ocg_hbm, ocb_hbm, eg_hbm, eb_hbm,
             out_hbm,
             idx_v, ob_v, icv_v, ioc_v, ipos_v, sel_v,
             cv_rows, oc_rows, pos_rows, out_rows,
             cvg_v, cvb_v, ocg_v, ocb_v, eg_v, eb_v, ty0_v,
             sem0, sem1, sem2):
    wid = lax.axis_index("s") * 2 + lax.axis_index("c")
    base = wid * _RPW

    pltpu.sync_copy(idx_hbm.at[pl.ds(base, _RPW)], idx_v)
    pltpu.sync_copy(obase_hbm.at[pl.ds(base, _RPW)], ob_v)
    pltpu.sync_copy(ipos_hbm.at[pl.ds(base, _RPW)], ipos_v)
    pltpu.sync_copy(cvg_hbm, cvg_v)
    pltpu.sync_copy(cvb_hbm, cvb_v)
    pltpu.sync_copy(ocg_hbm, ocg_v)
    pltpu.sync_copy(ocb_hbm, ocb_v)
    pltpu.sync_copy(eg_hbm, eg_v)
    pltpu.sync_copy(eb_hbm, eb_v)
    pltpu.sync_copy(type_hbm.at[0], ty0_v)

    for j in range(_RPW // _LANE):
        sl = pl.ds(j * _LANE, _LANE)
        v = idx_v[sl]
        is_cv = v < _V
        icv_v[sl] = jnp.where(is_cv, v, 0)
        ioc_v[sl] = jnp.where(is_cv, 0, ob_v[sl] + (v - _V))
        sel_v[sl] = jnp.where(is_cv, jnp.float32(1.0), jnp.float32(0.0))

    c1 = pltpu.async_copy(cv_hbm.at[icv_v], cv_rows, sem0)
    c2 = pltpu.async_copy(ocr_hbm.at[ioc_v], oc_rows, sem1)
    c3 = pltpu.async_copy(pos_hbm.at[ipos_v], pos_rows, sem2)
    c1.wait()
    c2.wait()
    c3.wait()

    def one_row(r):
        rb = jnp.full((_LANE,), r, dtype=jnp.int32)
        sel = plsc.load_gather(sel_v, [rb])
        inv = 1.0 - sel
        xs = []
        ps = []
        for h in range(_H // _LANE):
            hs = pl.ds(h * _LANE, _LANE)
            xs.append(cv_rows[r, hs] * sel + oc_rows[r, hs] * inv)
            ps.append(pos_rows[r, hs] + ty0_v[hs])
        sx = (xs[0] + xs[1]) + (xs[2] + xs[3])
        qx = (xs[0] * xs[0] + xs[1] * xs[1]) + (xs[2] * xs[2] + xs[3] * xs[3])
        sp = (ps[0] + ps[1]) + (ps[2] + ps[3])
        qp = (ps[0] * ps[0] + ps[1] * ps[1]) + (ps[2] * ps[2] + ps[3] * ps[3])
        mu = jnp.sum(sx) * (1.0 / _H)
        mup = jnp.sum(sp) * (1.0 / _H)
        var = jnp.sum(qx) * (1.0 / _H) - mu * mu
        varp = jnp.sum(qp) * (1.0 / _H) - mup * mup
        rstd = _rsqrt16(var + _EPS)
        rstdp = _rsqrt16(varp + _EPS)
        for h in range(_H // _LANE):
            hs = pl.ds(h * _LANE, _LANE)
            g = (cvg_v[hs] * sel + ocg_v[hs] * inv) * rstd
            be = cvb_v[hs] * sel + ocb_v[hs] * inv
            out_rows[r, hs] = ((xs[h] - mu) * g + be
                               + (ps[h] - mup) * rstdp * eg_v[hs] + eb_v[hs])

    def row_body(i, carry):
        one_row(2 * i)
        one_row(2 * i + 1)
        return carry

    lax.fori_loop(0, _RPW // 2, row_body, 0)
    pltpu.sync_copy(out_rows, out_hbm.at[pl.ds(base, _RPW)])


@functools.partial(
    pl.kernel,
    mesh=plsc.VectorSubcoreMesh(core_axis_name="c", subcore_axis_name="s"),
    out_type=jax.ShapeDtypeStruct((_TOT, _H), jnp.float32),
    compiler_params=pltpu.CompilerParams(
        use_tc_tiling_on_sc=False, needs_layout_passes=False),
    scratch_types=[
        pltpu.VMEM((_RPW,), jnp.int32),
        pltpu.VMEM((_RPW,), jnp.int32),
        pltpu.VMEM((_RPW,), jnp.int32),
        pltpu.VMEM((_RPW,), jnp.int32),
        pltpu.VMEM((_RPW,), jnp.int32),
        pltpu.VMEM((_RPW,), jnp.float32),
        pltpu.VMEM((_RPW, _H), jnp.float32),
        pltpu.VMEM((_RPW, _H), jnp.float32),
        pltpu.VMEM((_RPW, _H), jnp.float32),
        pltpu.VMEM((_RPW, _H), jnp.float32),
        pltpu.VMEM((_H,), jnp.float32),
        pltpu.VMEM((_H,), jnp.float32),
        pltpu.VMEM((_H,), jnp.float32),
        pltpu.VMEM((_H,), jnp.float32),
        pltpu.VMEM((_H,), jnp.float32),
        pltpu.VMEM((_H,), jnp.float32),
        pltpu.VMEM((_H,), jnp.float32),
        pltpu.SemaphoreType.DMA,
        pltpu.SemaphoreType.DMA,
        pltpu.SemaphoreType.DMA,
    ],
)
def _prev_embed_sc(idx_hbm, obase_hbm, ipos_hbm, cv_hbm, ocr_hbm, pos_hbm,
                   type_hbm, cvg_hbm, cvb_hbm, ocg_hbm, ocb_hbm, eg_hbm,
                   eb_hbm, out_hbm, *scratch):
    _sc_body(idx_hbm, obase_hbm, ipos_hbm, cv_hbm, ocr_hbm, pos_hbm, type_hbm,
             cvg_hbm, cvb_hbm, ocg_hbm, ocb_hbm, eg_hbm, eb_hbm,
             out_hbm, *scratch)


def kernel(encoder_input_embed, encoder_input_mask, ocr_emb, common_voc_emb,
           prev_inds, pos_emb, type_emb, cv_gamma, cv_beta, ocr_gamma,
           ocr_beta, emb_gamma, emb_beta):
    b, l = prev_inds.shape
    idx_flat = prev_inds.reshape(-1).astype(jnp.int32)
    idx_pad = jnp.zeros((_TOT,), jnp.int32).at[: b * l].set(idx_flat)
    flat = jnp.arange(_TOT, dtype=jnp.int32)
    obase = (flat // _L) * _OCR
    ipos = jnp.remainder(flat, _L)
    ocr_flat = ocr_emb.reshape(-1, _H)

    prev = _prev_embed_sc(idx_pad, obase, ipos, common_voc_emb, ocr_flat,
                          pos_emb, type_emb, cv_gamma, cv_beta, ocr_gamma,
                          ocr_beta, emb_gamma, emb_beta)
    prev_embed = prev[: b * l].reshape(b, l, _H)

    encoder_inputs = jnp.concatenate([encoder_input_embed, prev_embed], axis=1)
    encoder_inputs_mask = jnp.concatenate(
        [encoder_input_mask, jnp.zeros((b, l), jnp.float32)], axis=1)
    ext = jnp.full((b, 1, l, l), -10000.0, jnp.float32)
    return (encoder_inputs, encoder_inputs_mask, ext)
